# baseline (speedup 1.0000x reference)
import jax
import jax.numpy as jnp
from jax.experimental import pallas as pl
from jax.experimental.pallas import tpu as pltpu

_TOK_BLOCK = 512


def _router_kernel(x0_ref, x1_ref, x2_ref, x3_ref, w_ref, b_ref,
                   probs_ref, idx_ref, wts_ref):
    def half(x):
        lt = jax.lax.dot_general(
            w_ref[...], x, (((1,), (1,)), ((), ())),
            preferred_element_type=jnp.float32,
        )
        lt = lt + b_ref[...]
        rows = jax.lax.broadcasted_iota(jnp.int32, lt.shape, 0)
        n = lt.shape[0]
        v1 = jnp.max(lt, axis=0, keepdims=True)
        i1 = jnp.min(jnp.where(lt == v1, rows, n), axis=0, keepdims=True)
        masked = jnp.where(rows == i1, -jnp.inf, lt)
        v2 = jnp.max(masked, axis=0, keepdims=True)
        i2 = jnp.min(jnp.where(masked == v2, rows, n), axis=0, keepdims=True)
        w1 = 1.0 / (1.0 + jnp.exp(v2 - v1))
        e = jnp.exp(lt - v1)
        z = jnp.sum(e, axis=0, keepdims=True)
        probs = jnp.transpose(e / z)
        idx = jnp.concatenate([i1, i2], axis=0)
        wts = jnp.concatenate([w1, 1.0 - w1], axis=0)
        return probs, idx, wts

    for k, xref in enumerate((x0_ref, x1_ref, x2_ref, x3_ref)):
        p, ix, wt = half(xref[0])
        probs_ref[k] = p
        idx_ref[k] = ix
        wts_ref[k] = wt


@jax.jit
def kernel(x, W, b):
    n_tok, d_model = x.shape
    n_exp = W.shape[0]
    t = _TOK_BLOCK
    h = n_tok // 4
    xr = x.reshape(4, h, d_model)
    probs, idx_t, wts_t = pl.pallas_call(
        _router_kernel,
        grid=(h // t,),
        in_specs=[
            pl.BlockSpec((1, t, d_model), lambda i: (0, i, 0)),
            pl.BlockSpec((1, t, d_model), lambda i: (1, i, 0)),
            pl.BlockSpec((1, t, d_model), lambda i: (2, i, 0)),
            pl.BlockSpec((1, t, d_model), lambda i: (3, i, 0)),
            pl.BlockSpec((n_exp, d_model), lambda i: (0, 0)),
            pl.BlockSpec((n_exp, 1), lambda i: (0, 0)),
        ],
        out_specs=[
            pl.BlockSpec((4, t, n_exp), lambda i: (0, i, 0)),
            pl.BlockSpec((4, 2, t), lambda i: (0, 0, i)),
            pl.BlockSpec((4, 2, t), lambda i: (0, 0, i)),
        ],
        out_shape=[
            jax.ShapeDtypeStruct((4, h, n_exp), jnp.float32),
            jax.ShapeDtypeStruct((4, 2, h), jnp.int32),
            jax.ShapeDtypeStruct((4, 2, h), jnp.float32),
        ],
        compiler_params=pltpu.CompilerParams(
            dimension_semantics=("parallel",),
        ),
    )(xr, xr, xr, xr, W.reshape(n_exp, d_model), b.reshape(n_exp, 1))
    idx = jnp.transpose(idx_t, (0, 2, 1)).reshape(n_tok, 2)
    wts = jnp.transpose(wts_t, (0, 2, 1)).reshape(n_tok, 2)
    return probs.reshape(n_tok, n_exp), idx, wts
